# baseline (speedup 1.0000x reference)
import functools

import jax
import jax.numpy as jnp
import numpy as np
from jax import lax
from jax.experimental import pallas as pl
from jax.experimental.pallas import tpu as pltpu
from jax.experimental.pallas import tpu_sc as plsc

N = 10000
E = 320000
D = 16

NC = 2
NS = 16
NW = NC * NS

BATCH = 80
ROWS = E // BATCH
RPW = ROWS // NW
K = 25
ITERS = RPW // K
CHUNK = K * BATCH
NCHUNK = ROWS // K
NPAD = 10240
NPT = NPAD // NS

@functools.cache
def _sc_kernels():
    mesh = plsc.VectorSubcoreMesh(
        core_axis_name="c", subcore_axis_name="s", num_cores=NC, num_subcores=NS
    )

    @functools.partial(
        pl.kernel,
        out_type=jax.ShapeDtypeStruct((E, D), jnp.float32),
        mesh=mesh,
        scratch_types=[
            pltpu.VMEM((K, BATCH), jnp.int32),
            pltpu.VMEM((CHUNK, D), jnp.float32),
            pltpu.SemaphoreType.DMA,
        ],
        compiler_params=pltpu.CompilerParams(use_tc_tiling_on_sc=False),
    )
    def _sc_gather(x_hbm, ei_hbm, xj_hbm, idx_v, rows_v, sem):
        wid = lax.axis_index("s") * NC + lax.axis_index("c")
        for it in range(ITERS):
            chunk = wid * ITERS + it
            pltpu.sync_copy(ei_hbm.at[0, chunk], idx_v)
            descs = [
                pltpu.async_copy(
                    x_hbm.at[idx_v.at[j]], rows_v.at[pl.ds(j * BATCH, BATCH)], sem
                )
                for j in range(K)
            ]
            for d in descs:
                d.wait()
            pltpu.sync_copy(rows_v, xj_hbm.at[pl.ds(chunk * CHUNK, CHUNK)])

    @functools.partial(
        pl.kernel,
        out_type=jax.ShapeDtypeStruct((NC, NPAD, D), jnp.float32),
        mesh=mesh,
        scratch_types=[
            pltpu.VMEM((K, BATCH), jnp.int32),
            pltpu.VMEM((CHUNK, D), jnp.float32),
            pltpu.VMEM((NPT, D), jnp.float32),
            pltpu.VMEM_SHARED((NPAD, D), jnp.float32),
            pltpu.SemaphoreType.DMA,
        ],
        compiler_params=pltpu.CompilerParams(use_tc_tiling_on_sc=False),
    )
    def _sc_scatter(msg_hbm, ei_hbm, out_hbm, idx_v, rows_v, zbuf_v, acc_sh, sem):
        c = lax.axis_index("c")
        s = lax.axis_index("s")
        wid = s * NC + c

        def _zero(i, carry):
            zbuf_v[i] = jnp.zeros((D,), jnp.float32)
            return carry

        lax.fori_loop(0, NPT, _zero, 0)
        pltpu.sync_copy(zbuf_v, acc_sh.at[pl.ds(s * NPT, NPT)])
        plsc.subcore_barrier()

        for it in range(ITERS):
            chunk = wid * ITERS + it
            pltpu.sync_copy(ei_hbm.at[1, chunk], idx_v)
            pltpu.sync_copy(msg_hbm.at[pl.ds(chunk * CHUNK, CHUNK)], rows_v)
            descs = [
                pltpu.async_copy(
                    rows_v.at[pl.ds(j * BATCH, BATCH)],
                    acc_sh.at[idx_v.at[j]],
                    sem,
                    add=True,
                )
                for j in range(K)
            ]
            for d in descs:
                d.wait()

        plsc.subcore_barrier()
        pltpu.sync_copy(
            acc_sh.at[pl.ds(s * NPT, NPT)], out_hbm.at[c, pl.ds(s * NPT, NPT)]
        )

    return _sc_gather, _sc_scatter


P = 8
E8 = E // P
BE = 1000
W2 = P * D * D


def _tc_msg_body(xj_ref, ea_ref, wetb_ref, repb_ref, foldb_ref, msg_ref):
    ea_p = ea_ref[...].astype(jnp.bfloat16)
    xj_p = xj_ref[...].astype(jnp.bfloat16)
    ef = jnp.dot(ea_p, wetb_ref[...], preferred_element_type=jnp.float32)
    xr = jnp.dot(xj_p, repb_ref[...], preferred_element_type=jnp.float32)
    pr = (xr * ef).astype(jnp.bfloat16)
    msg_ref[...] = jnp.dot(pr, foldb_ref[...], preferred_element_type=jnp.float32)


_tc_msg = pl.pallas_call(
    _tc_msg_body,
    grid=(E8 // BE,),
    in_specs=[
        pl.BlockSpec((BE, P * D), lambda i: (i, 0)),
        pl.BlockSpec((BE, P * D), lambda i: (i, 0)),
        pl.BlockSpec((P * D, W2), lambda i: (0, 0)),
        pl.BlockSpec((P * D, W2), lambda i: (0, 0)),
        pl.BlockSpec((W2, P * D), lambda i: (0, 0)),
    ],
    out_specs=pl.BlockSpec((BE, P * D), lambda i: (i, 0)),
    out_shape=jax.ShapeDtypeStruct((E8, P * D), jnp.float32),
)

_REP = np.repeat(np.eye(D, dtype=np.float32), D, axis=1)
_FOLD = np.tile(np.eye(D, dtype=np.float32), (D, 1))
_EYE8 = np.eye(P, dtype=np.float32)
_REPB = np.kron(_EYE8, _REP)
_FOLDB = np.kron(_EYE8, _FOLD)


N8 = N // P
NPAD8 = NPAD // P


def _tc_out_body(p_ref, x_ref, wntb_ref, o_ref):
    xw = jnp.dot(x_ref[...], wntb_ref[...], preferred_element_type=jnp.float32)
    o_ref[...] = p_ref[0, :N8] + p_ref[1, :N8] + xw


_tc_out = pl.pallas_call(
    _tc_out_body,
    out_shape=jax.ShapeDtypeStruct((N8, P * D), jnp.float32),
)


def kernel(x, edge_index, edge_attr, W_edge, W_node):
    sc_gather, sc_scatter = _sc_kernels()
    ei4d = edge_index.reshape(2, NCHUNK, K, BATCH)
    xj = sc_gather(x, ei4d)
    bf = jnp.bfloat16
    wetb = jnp.kron(jnp.eye(P, dtype=jnp.float32), W_edge.T).astype(bf)
    msg_p = _tc_msg(
        xj.reshape(E8, P * D),
        edge_attr.reshape(E8, P * D),
        wetb,
        jnp.asarray(_REPB.astype(np.float32)).astype(bf),
        jnp.asarray(_FOLDB).astype(bf),
    )
    partials = sc_scatter(msg_p.reshape(E, D), ei4d)
    wntb = jnp.kron(jnp.eye(P, dtype=jnp.float32), W_node.T)
    out_p = _tc_out(partials.reshape(NC, NPAD8, P * D), x.reshape(N8, P * D), wntb)
    return out_p.reshape(N, D)


# speedup vs baseline: 1.2375x; 1.2375x over previous
import functools

import jax
import jax.numpy as jnp
import numpy as np
from jax import lax
from jax.experimental import pallas as pl
from jax.experimental.pallas import tpu as pltpu
from jax.experimental.pallas import tpu_sc as plsc

N = 10000
E = 320000
D = 16

NC = 2
NS = 16
NW = NC * NS

BATCH = 80
ROWS = E // BATCH
RPW = ROWS // NW
K = 25
ITERS = RPW // K
CHUNK = K * BATCH
NCHUNK = ROWS // K
NPAD = 10240
NPT = NPAD // NS

@functools.cache
def _sc_kernels():
    mesh = plsc.VectorSubcoreMesh(
        core_axis_name="c", subcore_axis_name="s", num_cores=NC, num_subcores=NS
    )

    @functools.partial(
        pl.kernel,
        out_type=jax.ShapeDtypeStruct((E, D), jnp.float32),
        mesh=mesh,
        scratch_types=[
            pltpu.VMEM((K, BATCH), jnp.int32),
            pltpu.VMEM((CHUNK, D), jnp.float32),
            pltpu.SemaphoreType.DMA,
            pltpu.SemaphoreType.DMA,
        ],
        compiler_params=pltpu.CompilerParams(use_tc_tiling_on_sc=False),
    )
    def _sc_gather(x_hbm, src_hbm, xj_hbm, idx_v, rows_v, sem, isem):
        wid = lax.axis_index("s") * NC + lax.axis_index("c")
        for it in range(ITERS):
            chunk = wid * ITERS + it
            base = chunk * CHUNK
            idescs = [
                pltpu.async_copy(
                    src_hbm.at[pl.ds(base + j * BATCH, BATCH)], idx_v.at[j], isem
                )
                for j in range(K)
            ]
            for d in idescs:
                d.wait()
            descs = [
                pltpu.async_copy(
                    x_hbm.at[idx_v.at[j]], rows_v.at[pl.ds(j * BATCH, BATCH)], sem
                )
                for j in range(K)
            ]
            for d in descs:
                d.wait()
            pltpu.sync_copy(rows_v, xj_hbm.at[pl.ds(chunk * CHUNK, CHUNK)])

    @functools.partial(
        pl.kernel,
        out_type=jax.ShapeDtypeStruct((NC, NPAD, D), jnp.float32),
        mesh=mesh,
        scratch_types=[
            pltpu.VMEM((K, BATCH), jnp.int32),
            pltpu.VMEM((CHUNK, D), jnp.float32),
            pltpu.VMEM((NPT, D), jnp.float32),
            pltpu.VMEM_SHARED((NPAD, D), jnp.float32),
            pltpu.SemaphoreType.DMA,
            pltpu.SemaphoreType.DMA,
        ],
        compiler_params=pltpu.CompilerParams(use_tc_tiling_on_sc=False),
    )
    def _sc_scatter(msg_hbm, dst_hbm, out_hbm, idx_v, rows_v, zbuf_v, acc_sh, sem, isem):
        c = lax.axis_index("c")
        s = lax.axis_index("s")
        wid = s * NC + c

        def _zero(i, carry):
            zbuf_v[i] = jnp.zeros((D,), jnp.float32)
            return carry

        lax.fori_loop(0, NPT, _zero, 0)
        pltpu.sync_copy(zbuf_v, acc_sh.at[pl.ds(s * NPT, NPT)])
        plsc.subcore_barrier()

        for it in range(ITERS):
            chunk = wid * ITERS + it
            base = chunk * CHUNK
            idescs = [
                pltpu.async_copy(
                    dst_hbm.at[pl.ds(base + j * BATCH, BATCH)], idx_v.at[j], isem
                )
                for j in range(K)
            ]
            pltpu.sync_copy(msg_hbm.at[pl.ds(base, CHUNK)], rows_v)
            for d in idescs:
                d.wait()
            descs = [
                pltpu.async_copy(
                    rows_v.at[pl.ds(j * BATCH, BATCH)],
                    acc_sh.at[idx_v.at[j]],
                    sem,
                    add=True,
                )
                for j in range(K)
            ]
            for d in descs:
                d.wait()

        plsc.subcore_barrier()
        pltpu.sync_copy(
            acc_sh.at[pl.ds(s * NPT, NPT)], out_hbm.at[c, pl.ds(s * NPT, NPT)]
        )

    return _sc_gather, _sc_scatter


P = 8
E8 = E // P
BE = 1000
W2 = P * D * D


def _tc_msg_body(xj_ref, ea_ref, we_ref, foldt_ref, msg_ref):
    xjt = jnp.transpose(xj_ref[...]).astype(jnp.bfloat16)
    eat = jnp.transpose(ea_ref[...])
    cols = []
    for g in range(P):
        ea_g = eat[D * g : D * (g + 1), :]
        xj_g = xjt[D * g : D * (g + 1), :]
        ef_g = jnp.dot(
            we_ref[...], ea_g, preferred_element_type=jnp.float32
        ).astype(jnp.bfloat16)
        xr_g = jnp.broadcast_to(
            xj_g[:, None, :], (D, D, BE)
        ).reshape(D * D, BE)
        cols.append(
            jnp.dot(
                foldt_ref[...], xr_g * ef_g,
                preferred_element_type=jnp.float32,
            )
        )
    msg_ref[...] = jnp.transpose(jnp.concatenate(cols, axis=0))


_tc_msg = pl.pallas_call(
    _tc_msg_body,
    grid=(E8 // BE,),
    in_specs=[
        pl.BlockSpec((BE, P * D), lambda i: (i, 0)),
        pl.BlockSpec((BE, P * D), lambda i: (i, 0)),
        pl.BlockSpec((D * D, D), lambda i: (0, 0)),
        pl.BlockSpec((D, D * D), lambda i: (0, 0)),
    ],
    out_specs=pl.BlockSpec((BE, P * D), lambda i: (i, 0)),
    out_shape=jax.ShapeDtypeStruct((E8, P * D), jnp.float32),
)

_FOLDT = np.tile(np.eye(D, dtype=np.float32), (D, 1)).T


N8 = N // P
NPAD8 = NPAD // P


def _tc_out_body(p_ref, x_ref, wntb_ref, o_ref):
    xw = jnp.dot(x_ref[...], wntb_ref[...], preferred_element_type=jnp.float32)
    o_ref[...] = p_ref[0, :N8] + p_ref[1, :N8] + xw


_tc_out = pl.pallas_call(
    _tc_out_body,
    out_shape=jax.ShapeDtypeStruct((N8, P * D), jnp.float32),
)


def kernel(x, edge_index, edge_attr, W_edge, W_node):
    sc_gather, sc_scatter = _sc_kernels()
    xj = sc_gather(x, edge_index[0])
    bf = jnp.bfloat16
    msg_p = _tc_msg(
        xj.reshape(E8, P * D),
        edge_attr.astype(bf).reshape(E8, P * D),
        W_edge.astype(bf),
        jnp.asarray(_FOLDT).astype(bf),
    )
    partials = sc_scatter(msg_p.reshape(E, D), edge_index[1])
    wntb = jnp.kron(jnp.eye(P, dtype=jnp.float32), W_node.T)
    out_p = _tc_out(partials.reshape(NC, NPAD8, P * D), x.reshape(N8, P * D), wntb)
    return out_p.reshape(N, D)


# speedup vs baseline: 1.2603x; 1.0184x over previous
import functools

import jax
import jax.numpy as jnp
import numpy as np
from jax import lax
from jax.experimental import pallas as pl
from jax.experimental.pallas import tpu as pltpu
from jax.experimental.pallas import tpu_sc as plsc

N = 10000
E = 320000
D = 16

NC = 2
NS = 16
NW = NC * NS

BATCH = 80
ROWS = E // BATCH
RPW = ROWS // NW
K = 25
ITERS = RPW // K
CHUNK = K * BATCH
NCHUNK = ROWS // K
NPAD = 10240
NPT = NPAD // NS

@functools.cache
def _sc_kernels():
    mesh = plsc.VectorSubcoreMesh(
        core_axis_name="c", subcore_axis_name="s", num_cores=NC, num_subcores=NS
    )

    @functools.partial(
        pl.kernel,
        out_type=jax.ShapeDtypeStruct((E, D), jnp.float32),
        mesh=mesh,
        scratch_types=[
            pltpu.VMEM((K, BATCH), jnp.int32),
            pltpu.VMEM((CHUNK, D), jnp.float32),
            pltpu.SemaphoreType.DMA,
            pltpu.SemaphoreType.DMA,
        ],
        compiler_params=pltpu.CompilerParams(use_tc_tiling_on_sc=False),
    )
    def _sc_gather(x_hbm, ei_hbm, xj_hbm, idx_v, rows_v, sem, isem):
        wid = lax.axis_index("s") * NC + lax.axis_index("c")
        for it in range(ITERS):
            chunk = wid * ITERS + it
            base = chunk * CHUNK
            idescs = [
                pltpu.async_copy(
                    ei_hbm.at[0, pl.ds(base + j * BATCH, BATCH)], idx_v.at[j], isem
                )
                for j in range(K)
            ]
            for d in idescs:
                d.wait()
            descs = [
                pltpu.async_copy(
                    x_hbm.at[idx_v.at[j]], rows_v.at[pl.ds(j * BATCH, BATCH)], sem
                )
                for j in range(K)
            ]
            for d in descs:
                d.wait()
            pltpu.sync_copy(rows_v, xj_hbm.at[pl.ds(chunk * CHUNK, CHUNK)])

    @functools.partial(
        pl.kernel,
        out_type=jax.ShapeDtypeStruct((NC, NPAD, D), jnp.float32),
        mesh=mesh,
        scratch_types=[
            pltpu.VMEM((K, BATCH), jnp.int32),
            pltpu.VMEM((CHUNK, D), jnp.float32),
            pltpu.VMEM((NPT, D), jnp.float32),
            pltpu.VMEM_SHARED((NPAD, D), jnp.float32),
            pltpu.SemaphoreType.DMA,
            pltpu.SemaphoreType.DMA,
        ],
        compiler_params=pltpu.CompilerParams(use_tc_tiling_on_sc=False),
    )
    def _sc_scatter(msg_hbm, ei_hbm, out_hbm, idx_v, rows_v, zbuf_v, acc_sh, sem, isem):
        c = lax.axis_index("c")
        s = lax.axis_index("s")
        wid = s * NC + c

        def _zero(i, carry):
            zbuf_v[i] = jnp.zeros((D,), jnp.float32)
            return carry

        lax.fori_loop(0, NPT, _zero, 0)
        pltpu.sync_copy(zbuf_v, acc_sh.at[pl.ds(s * NPT, NPT)])
        plsc.subcore_barrier()

        for it in range(ITERS):
            chunk = wid * ITERS + it
            base = chunk * CHUNK
            idescs = [
                pltpu.async_copy(
                    ei_hbm.at[1, pl.ds(base + j * BATCH, BATCH)], idx_v.at[j], isem
                )
                for j in range(K)
            ]
            pltpu.sync_copy(msg_hbm.at[pl.ds(base, CHUNK)], rows_v)
            for d in idescs:
                d.wait()
            descs = [
                pltpu.async_copy(
                    rows_v.at[pl.ds(j * BATCH, BATCH)],
                    acc_sh.at[idx_v.at[j]],
                    sem,
                    add=True,
                )
                for j in range(K)
            ]
            for d in descs:
                d.wait()

        plsc.subcore_barrier()
        pltpu.sync_copy(
            acc_sh.at[pl.ds(s * NPT, NPT)], out_hbm.at[c, pl.ds(s * NPT, NPT)]
        )

    return _sc_gather, _sc_scatter


P = 8
E8 = E // P
BE = 1000
W2 = P * D * D


def _tc_msg_body(xj_ref, ea_ref, we_ref, foldt_ref, msg_ref):
    xjt = jnp.transpose(xj_ref[...]).astype(jnp.bfloat16)
    eat = jnp.transpose(ea_ref[...])
    cols = []
    for g in range(P):
        ea_g = eat[D * g : D * (g + 1), :]
        xj_g = xjt[D * g : D * (g + 1), :]
        ef_g = jnp.dot(
            we_ref[...], ea_g, preferred_element_type=jnp.float32
        ).astype(jnp.bfloat16)
        xr_g = jnp.broadcast_to(
            xj_g[:, None, :], (D, D, BE)
        ).reshape(D * D, BE)
        cols.append(
            jnp.dot(
                foldt_ref[...], xr_g * ef_g,
                preferred_element_type=jnp.float32,
            )
        )
    msg_ref[...] = jnp.transpose(jnp.concatenate(cols, axis=0))


_tc_msg = pl.pallas_call(
    _tc_msg_body,
    grid=(E8 // BE,),
    in_specs=[
        pl.BlockSpec((BE, P * D), lambda i: (i, 0)),
        pl.BlockSpec((BE, P * D), lambda i: (i, 0)),
        pl.BlockSpec((D * D, D), lambda i: (0, 0)),
        pl.BlockSpec((D, D * D), lambda i: (0, 0)),
    ],
    out_specs=pl.BlockSpec((BE, P * D), lambda i: (i, 0)),
    out_shape=jax.ShapeDtypeStruct((E8, P * D), jnp.float32),
)

_FOLDT = np.tile(np.eye(D, dtype=np.float32), (D, 1)).T


N8 = N // P
NPAD8 = NPAD // P


def _tc_out_body(p_ref, x_ref, wntb_ref, o_ref):
    xw = jnp.dot(x_ref[...], wntb_ref[...], preferred_element_type=jnp.float32)
    o_ref[...] = p_ref[0, :N8] + p_ref[1, :N8] + xw


_tc_out = pl.pallas_call(
    _tc_out_body,
    out_shape=jax.ShapeDtypeStruct((N8, P * D), jnp.float32),
)


def kernel(x, edge_index, edge_attr, W_edge, W_node):
    sc_gather, sc_scatter = _sc_kernels()
    xj = sc_gather(x, edge_index)
    bf = jnp.bfloat16
    msg_p = _tc_msg(
        xj.reshape(E8, P * D),
        edge_attr.astype(bf).reshape(E8, P * D),
        W_edge.astype(bf),
        jnp.asarray(_FOLDT).astype(bf),
    )
    partials = sc_scatter(msg_p.reshape(E, D), edge_index)
    wntb = jnp.kron(jnp.eye(P, dtype=jnp.float32), W_node.T)
    out_p = _tc_out(partials.reshape(NC, NPAD8, P * D), x.reshape(N8, P * D), wntb)
    return out_p.reshape(N, D)
